# baseline (speedup 1.0000x reference)
import functools

import jax
import jax.numpy as jnp
from jax import lax
from jax.experimental import pallas as pl
from jax.experimental.pallas import tpu as pltpu
from jax.experimental.pallas import tpu_sc as plsc

_USER_NUM = 1000000
_NC = 2
_NS = 16
_NW = _NC * _NS
_L = 16
_PACK = 8
_CHUNK = 128


def _ffm_body(user_hbm, item_hbm, fc_hbm, bias_hbm, emb0_hbm, emb1_hbm,
              out_hbm, u_idx, i_idx, uq, iq, us, isub, rows_u, rows_i,
              fc_u, fc_i, bias_v, out_v, sem_u, sem_i, sem_g, b_per_w):
    wid = lax.axis_index("s") * _NC + lax.axis_index("c")
    base = wid * b_per_w
    n_chunks = b_per_w // _CHUNK

    cp_u = pltpu.async_copy(user_hbm.at[pl.ds(base, b_per_w)], u_idx, sem_u)
    cp_i = pltpu.async_copy(item_hbm.at[pl.ds(base, b_per_w)], i_idx, sem_i)
    pltpu.sync_copy(bias_hbm, bias_v)

    cp_u.wait()
    for v in range(b_per_w // _L):
        sl = pl.ds(v * _L, _L)
        u = u_idx[sl]
        uq[sl] = lax.shift_right_logical(u, 3)
        us[sl] = lax.shift_left(jnp.bitwise_and(u, 7), 4)
    cp_i.wait()
    for v in range(b_per_w // _L):
        sl = pl.ds(v * _L, _L)
        it = i_idx[sl] + _USER_NUM
        i_idx[sl] = it
        iq[sl] = lax.shift_right_logical(it, 3)
        isub[sl] = lax.shift_left(jnp.bitwise_and(it, 7), 4)

    fc_gathers = []
    for c in range(n_chunks):
        sl = pl.ds(c * _CHUNK, _CHUNK)
        fc_gathers.append(pltpu.async_copy(fc_hbm.at[u_idx.at[sl]],
                                           fc_u.at[sl], sem_g))
        fc_gathers.append(pltpu.async_copy(fc_hbm.at[i_idx.at[sl]],
                                           fc_i.at[sl], sem_g))

    iota = lax.iota(jnp.int32, _L)
    bias_bc = bias_v[...]

    for c in range(n_chunks):
        sl = pl.ds(c * _CHUNK, _CHUNK)
        gu = pltpu.async_copy(emb1_hbm.at[uq.at[sl]], rows_u, sem_u)
        gi = pltpu.async_copy(emb0_hbm.at[iq.at[sl]], rows_i, sem_i)
        if c == 0:
            for g in fc_gathers:
                g.wait()
        gu.wait()
        gi.wait()
        for g in range(_CHUNK // _L):
            off = c * _CHUNK + g * _L
            gsl = pl.ds(off, _L)
            rid = g * _L + iota
            ub = us[gsl]
            ib = isub[gsl]
            acc = fc_u[gsl] + fc_i[gsl] + bias_bc
            for k in range(_L):
                a = plsc.load_gather(rows_u, [rid, ub + k])
                b = plsc.load_gather(rows_i, [rid, ib + k])
                acc = acc + a * b
            out_v[gsl] = acc

    pltpu.sync_copy(out_v, out_hbm.at[pl.ds(base, b_per_w)])


def kernel(user, item, features, fc, bias, emb0, emb1):
    del features
    b = user.shape[0]
    b_per_w = b // _NW
    emb0p = emb0.reshape(-1, _PACK * _L)
    emb1p = emb1.reshape(-1, _PACK * _L)
    mesh = plsc.VectorSubcoreMesh(core_axis_name="c", subcore_axis_name="s")
    run = pl.kernel(
        functools.partial(_ffm_body, b_per_w=b_per_w),
        out_type=jax.ShapeDtypeStruct((b,), jnp.float32),
        mesh=mesh,
        scratch_types=[
            pltpu.VMEM((b_per_w,), jnp.int32),
            pltpu.VMEM((b_per_w,), jnp.int32),
            pltpu.VMEM((b_per_w,), jnp.int32),
            pltpu.VMEM((b_per_w,), jnp.int32),
            pltpu.VMEM((b_per_w,), jnp.int32),
            pltpu.VMEM((b_per_w,), jnp.int32),
            pltpu.VMEM((_CHUNK, _PACK * _L), jnp.float32),
            pltpu.VMEM((_CHUNK, _PACK * _L), jnp.float32),
            pltpu.VMEM((b_per_w,), jnp.float32),
            pltpu.VMEM((b_per_w,), jnp.float32),
            pltpu.VMEM((_L,), jnp.float32),
            pltpu.VMEM((b_per_w,), jnp.float32),
            pltpu.SemaphoreType.DMA,
            pltpu.SemaphoreType.DMA,
            pltpu.SemaphoreType.DMA,
        ],
        compiler_params=pltpu.CompilerParams(
            needs_layout_passes=False, use_tc_tiling_on_sc=True),
    )
    bias16 = jnp.broadcast_to(bias, (_L,))
    return run(user, item, fc.reshape(-1), bias16, emb0p, emb1p)
